# baseline (speedup 1.0000x reference)
import functools

import jax
import jax.numpy as jnp
from jax import lax
from jax.experimental import pallas as pl
from jax.experimental.pallas import tpu as pltpu
from jax.experimental.pallas import tpu_sc as plsc

NJ = 2
LANE = 128


@functools.lru_cache(maxsize=None)
def _make_transpose(n, dm):
    info = plsc.get_sparse_core_info()
    nc, ns = info.num_cores, info.num_subcores
    nw = nc * ns
    nfull = n // LANE
    ntail = n - nfull * LANE
    per_w = nfull // nw
    rem = nfull - per_w * nw
    rpb = LANE * dm // LANE
    mesh = plsc.VectorSubcoreMesh(core_axis_name="c", subcore_axis_name="s")

    @functools.partial(
        pl.kernel,
        mesh=mesh,
        compiler_params=pltpu.CompilerParams(
            use_tc_tiling_on_sc=True, needs_layout_passes=False
        ),
        out_type=jax.ShapeDtypeStruct((n * dm // LANE, LANE), jnp.float32),
        scratch_types=[
            pltpu.VMEM((2, dm, LANE), jnp.float32),
            pltpu.VMEM((2, rpb, LANE), jnp.float32),
            pltpu.SemaphoreType.DMA,
            pltpu.SemaphoreType.DMA,
        ],
    )
    def sc_transpose(vt_hbm, wtail_hbm, out_hbm, vb_v, t_v, sem_r, sem_w):
        wid = lax.axis_index("s") * nc + lax.axis_index("c")
        base = wid * per_w + jnp.minimum(wid, rem)
        nblk = per_w + (wid < rem).astype(jnp.int32)
        iota = lax.iota(jnp.int32, 16)

        def fire_read(c, p):
            pltpu.async_copy(
                vt_hbm.at[:, pl.ds(c * LANE, LANE)], vb_v.at[p], sem_r
            )

        def transpose_block(p, nrow):
            @plsc.parallel_loop(0, nrow, 1, unroll=4)
            def tr(i2):
                for h in range(8):
                    vals = plsc.load_gather(
                        vb_v.at[p],
                        [iota + (h % 2) * 16, jnp.zeros((16,), jnp.int32) + (i2 * 4 + h // 2)],
                    )
                    t_v[p, i2, pl.ds(h * 16, 16)] = vals

        fire_read(base, 0)

        def body(i, carry):
            p = i % 2

            @pl.when(i + 1 < nblk)
            def _():
                fire_read(base + i + 1, (i + 1) % 2)

            pltpu.make_async_copy(
                vt_hbm.at[:, pl.ds(0, LANE)], vb_v.at[p], sem_r
            ).wait()

            @pl.when(i >= 2)
            def _():
                pltpu.make_async_copy(
                    vt_hbm.at[:, pl.ds(0, LANE)], t_v.at[p], sem_w
                ).wait()

            transpose_block(p, rpb)
            pltpu.async_copy(
                t_v.at[p], out_hbm.at[pl.ds((base + i) * rpb, rpb)], sem_w
            )
            return carry

        lax.fori_loop(0, nblk, body, 0)
        for _ in range(2):
            pltpu.make_async_copy(
                vt_hbm.at[:, pl.ds(0, LANE)], t_v.at[0], sem_w
            ).wait()

        if ntail:

            @pl.when(wid == nw - 1)
            def _():
                trow = ntail * dm // LANE
                pltpu.sync_copy(wtail_hbm, t_v.at[0, pl.ds(0, trow)])
                pltpu.sync_copy(
                    t_v.at[0, pl.ds(0, trow)],
                    out_hbm.at[pl.ds(nfull * rpb, trow)],
                )

    return sc_transpose


@functools.lru_cache(maxsize=None)
def _make_kernel(b, s, dm):
    info = plsc.get_sparse_core_info()
    nc, ns = info.num_cores, info.num_subcores
    nw = nc * ns
    n_blocks = b // LANE
    cb_per_w = n_blocks // nw
    ngc = s // NJ
    ng = cb_per_w * ngc
    gr = dm // 8
    mesh = plsc.VectorSubcoreMesh(core_axis_name="c", subcore_axis_name="s")

    @functools.partial(
        pl.kernel,
        mesh=mesh,
        compiler_params=pltpu.CompilerParams(
            use_tc_tiling_on_sc=True, needs_layout_passes=False
        ),
        out_type=jax.ShapeDtypeStruct((s, gr, n_blocks, 8, LANE), jnp.float32),
        scratch_types=[
            pltpu.VMEM((2, NJ, LANE), jnp.int32),
            pltpu.VMEM((2, NJ, LANE), jnp.int32),
            pltpu.VMEM((2, NJ, LANE, LANE), jnp.float32),
            pltpu.VMEM((2, gr, NJ, 8, LANE), jnp.float32),
            pltpu.SemaphoreType.DMA,
            pltpu.SemaphoreType.DMA,
            pltpu.SemaphoreType.DMA,
        ],
    )
    def sc_gather(
        xt_hbm, w4_hbm, out_hbm, idx_v, idx4_v, rows_v, t_v, sem_a, sem_b, sem_w
    ):
        wid = lax.axis_index("s") * nc + lax.axis_index("c")
        c0 = wid * cb_per_w
        iota = lax.iota(jnp.int32, 16)
        sems = (sem_a, sem_b)

        def coords(g):
            cb = g // ngc
            q = g - cb * ngc
            return c0 + cb, q * NJ

        def stage_fire(g, par):
            cg, j0 = coords(g)
            pltpu.sync_copy(
                xt_hbm.at[pl.ds(j0, NJ), pl.ds(cg * LANE, LANE)], idx_v.at[par]
            )

            @plsc.parallel_loop(0, NJ * 8, 1, unroll=4)
            def mk4(k):
                u = k // 8
                m = k - u * 8
                v = idx_v[par, u, pl.ds(m * 16, 16)]
                idx4_v[par, u, pl.ds(m * 16, 16)] = lax.shift_right_logical(v, 2)

            for u in range(NJ):
                pltpu.async_copy(
                    w4_hbm.at[idx4_v.at[par, u]], rows_v.at[par, u], sems[par]
                )

        def drain_gather(par):
            for u in range(NJ):
                pltpu.make_async_copy(
                    w4_hbm.at[pl.ds(0, LANE)], rows_v.at[par, u], sems[par]
                ).wait()

        def drain_writes():
            for g_ in range(gr):
                pltpu.make_async_copy(
                    out_hbm.at[pl.ds(0, NJ), 0, 0], t_v.at[0, 0], sem_w
                ).wait()

        def process(g, par):
            cg, j0 = coords(g)

            @plsc.parallel_loop(0, NJ * 8, 1, unroll=4)
            def tr(k):
                u = k // 8
                m = k - u * 8
                rowv = iota + m * 16
                vi = idx_v[par, u, pl.ds(m * 16, 16)]
                offv = (vi & 3) * dm
                for gg in range(gr):
                    for ss in range(8):
                        vals = plsc.load_gather(
                            rows_v.at[par, u], [rowv, offv + (gg * 8 + ss)]
                        )
                        t_v[par, gg, u, ss, pl.ds(m * 16, 16)] = vals

            for gg in range(gr):
                pltpu.async_copy(
                    t_v.at[par, gg], out_hbm.at[pl.ds(j0, NJ), gg, cg], sem_w
                )

        stage_fire(0, 0)

        def body(s_, carry):
            stage_fire(2 * s_ + 1, 1)
            drain_gather(0)

            @pl.when(s_ > 0)
            def _():
                drain_writes()

            process(2 * s_, 0)
            stage_fire(2 * s_ + 2, 0)
            drain_gather(1)

            @pl.when(s_ > 0)
            def _():
                drain_writes()

            process(2 * s_ + 1, 1)
            return carry

        lax.fori_loop(0, ng // 2 - 1, body, 0)
        stage_fire(ng - 1, 1)
        drain_gather(0)
        drain_writes()
        process(ng - 2, 0)
        drain_gather(1)
        drain_writes()
        process(ng - 1, 1)
        drain_writes()
        drain_writes()

    return sc_gather


def kernel(x, W):
    b, s = x.shape
    dm = W.shape[1]
    xt = x.astype(jnp.int32).T
    nfull_cols = W.shape[0] // LANE * LANE
    wtail = W[nfull_cols:].reshape(-1, LANE)
    w4 = _make_transpose(W.shape[0], dm)(W.T, wtail)
    out = _make_kernel(b, s, dm)(xt, w4)
    return out.transpose(2, 4, 0, 1, 3).reshape(b, s, dm)


# speedup vs baseline: 1.0395x; 1.0395x over previous
import functools

import jax
import jax.numpy as jnp
from jax import lax
from jax.experimental import pallas as pl
from jax.experimental.pallas import tpu as pltpu
from jax.experimental.pallas import tpu_sc as plsc

NJ = 2
LANE = 128


@functools.lru_cache(maxsize=None)
def _make_transpose(n, dm):
    info = plsc.get_sparse_core_info()
    nc, ns = info.num_cores, info.num_subcores
    nw = nc * ns
    nfull = n // LANE
    ntail = n - nfull * LANE
    per_w = nfull // nw
    rem = nfull - per_w * nw
    rpb = LANE * dm // LANE
    mesh = plsc.VectorSubcoreMesh(core_axis_name="c", subcore_axis_name="s")

    @functools.partial(
        pl.kernel,
        mesh=mesh,
        compiler_params=pltpu.CompilerParams(
            use_tc_tiling_on_sc=True, needs_layout_passes=False
        ),
        out_type=jax.ShapeDtypeStruct((n * dm // LANE, LANE), jnp.float32),
        scratch_types=[
            pltpu.VMEM((2, dm, LANE), jnp.float32),
            pltpu.VMEM((2, rpb, LANE), jnp.float32),
            pltpu.SemaphoreType.DMA,
            pltpu.SemaphoreType.DMA,
        ],
    )
    def sc_transpose(vt_hbm, wtail_hbm, out_hbm, vb_v, t_v, sem_r, sem_w):
        wid = lax.axis_index("s") * nc + lax.axis_index("c")
        base = wid * per_w + jnp.minimum(wid, rem)
        nblk = per_w + (wid < rem).astype(jnp.int32)
        iota = lax.iota(jnp.int32, 16)

        def fire_read(c, p):
            pltpu.async_copy(
                vt_hbm.at[:, pl.ds(c * LANE, LANE)], vb_v.at[p], sem_r
            )

        zero = jnp.zeros((16,), jnp.int32)

        def transpose_block(p, nrow):
            @plsc.parallel_loop(0, nrow, 1, unroll=2)
            def tr(i2):
                b4 = zero + i2 * 4
                for h in range(8):
                    base_h = iota * LANE + ((h % 2) * 16 * LANE + h // 2)
                    vals = plsc.load_gather(vb_v.at[p], [zero, base_h + b4])
                    t_v[p, i2, pl.ds(h * 16, 16)] = vals

        fire_read(base, 0)

        def body(i, carry):
            p = i % 2

            @pl.when(i + 1 < nblk)
            def _():
                fire_read(base + i + 1, (i + 1) % 2)

            pltpu.make_async_copy(
                vt_hbm.at[:, pl.ds(0, LANE)], vb_v.at[p], sem_r
            ).wait()

            @pl.when(i >= 2)
            def _():
                pltpu.make_async_copy(
                    vt_hbm.at[:, pl.ds(0, LANE)], t_v.at[p], sem_w
                ).wait()

            transpose_block(p, rpb)
            pltpu.async_copy(
                t_v.at[p], out_hbm.at[pl.ds((base + i) * rpb, rpb)], sem_w
            )
            return carry

        lax.fori_loop(0, nblk, body, 0)
        for _ in range(2):
            pltpu.make_async_copy(
                vt_hbm.at[:, pl.ds(0, LANE)], t_v.at[0], sem_w
            ).wait()

        if ntail:

            @pl.when(wid == nw - 1)
            def _():
                trow = ntail * dm // LANE
                pltpu.sync_copy(wtail_hbm, t_v.at[0, pl.ds(0, trow)])
                pltpu.sync_copy(
                    t_v.at[0, pl.ds(0, trow)],
                    out_hbm.at[pl.ds(nfull * rpb, trow)],
                )

    return sc_transpose


@functools.lru_cache(maxsize=None)
def _make_kernel(b, s, dm):
    info = plsc.get_sparse_core_info()
    nc, ns = info.num_cores, info.num_subcores
    nw = nc * ns
    n_blocks = b // LANE
    cb_per_w = n_blocks // nw
    ngc = s // NJ
    ng = cb_per_w * ngc
    gr = dm // 8
    mesh = plsc.VectorSubcoreMesh(core_axis_name="c", subcore_axis_name="s")

    @functools.partial(
        pl.kernel,
        mesh=mesh,
        compiler_params=pltpu.CompilerParams(
            use_tc_tiling_on_sc=True, needs_layout_passes=False
        ),
        out_type=jax.ShapeDtypeStruct((s, gr, n_blocks, 8, LANE), jnp.float32),
        scratch_types=[
            pltpu.VMEM((2, NJ, LANE), jnp.int32),
            pltpu.VMEM((2, NJ, LANE), jnp.int32),
            pltpu.VMEM((2, NJ, LANE, LANE), jnp.float32),
            pltpu.VMEM((2, gr, NJ, 8, LANE), jnp.float32),
            pltpu.SemaphoreType.DMA,
            pltpu.SemaphoreType.DMA,
            pltpu.SemaphoreType.DMA,
        ],
    )
    def sc_gather(
        xt_hbm, w4_hbm, out_hbm, idx_v, idx4_v, rows_v, t_v, sem_a, sem_b, sem_w
    ):
        wid = lax.axis_index("s") * nc + lax.axis_index("c")
        c0 = wid * cb_per_w
        iota = lax.iota(jnp.int32, 16)
        sems = (sem_a, sem_b)

        def coords(g):
            cb = g // ngc
            q = g - cb * ngc
            return c0 + cb, q * NJ

        def stage_fire(g, par):
            cg, j0 = coords(g)
            pltpu.sync_copy(
                xt_hbm.at[pl.ds(j0, NJ), pl.ds(cg * LANE, LANE)], idx_v.at[par]
            )

            @plsc.parallel_loop(0, NJ * 8, 1, unroll=2)
            def mk4(k):
                u = k // 8
                m = k - u * 8
                v = idx_v[par, u, pl.ds(m * 16, 16)]
                idx4_v[par, u, pl.ds(m * 16, 16)] = lax.shift_right_logical(v, 2)

            for u in range(NJ):
                pltpu.async_copy(
                    w4_hbm.at[idx4_v.at[par, u]], rows_v.at[par, u], sems[par]
                )

        def drain_gather(par):
            for u in range(NJ):
                pltpu.make_async_copy(
                    w4_hbm.at[pl.ds(0, LANE)], rows_v.at[par, u], sems[par]
                ).wait()

        def drain_writes():
            for g_ in range(gr):
                pltpu.make_async_copy(
                    out_hbm.at[pl.ds(0, NJ), 0, 0], t_v.at[0, 0], sem_w
                ).wait()

        def process(g, par):
            cg, j0 = coords(g)

            zero = jnp.zeros((16,), jnp.int32)

            @plsc.parallel_loop(0, NJ * 8, 1, unroll=2)
            def tr(k):
                u = k // 8
                m = k - u * 8
                vi = idx_v[par, u, pl.ds(m * 16, 16)]
                base = (iota + m * 16) * LANE + (vi & 3) * dm
                for gg in range(gr):
                    for ss in range(8):
                        vals = plsc.load_gather(
                            rows_v.at[par, u], [zero, base + (gg * 8 + ss)]
                        )
                        t_v[par, gg, u, ss, pl.ds(m * 16, 16)] = vals

            for gg in range(gr):
                pltpu.async_copy(
                    t_v.at[par, gg], out_hbm.at[pl.ds(j0, NJ), gg, cg], sem_w
                )

        stage_fire(0, 0)

        def body(s_, carry):
            stage_fire(2 * s_ + 1, 1)
            drain_gather(0)

            @pl.when(s_ > 0)
            def _():
                drain_writes()

            process(2 * s_, 0)
            stage_fire(2 * s_ + 2, 0)
            drain_gather(1)

            @pl.when(s_ > 0)
            def _():
                drain_writes()

            process(2 * s_ + 1, 1)
            return carry

        lax.fori_loop(0, ng // 2 - 1, body, 0)
        stage_fire(ng - 1, 1)
        drain_gather(0)
        drain_writes()
        process(ng - 2, 0)
        drain_gather(1)
        drain_writes()
        process(ng - 1, 1)
        drain_writes()
        drain_writes()

    return sc_gather


def kernel(x, W):
    b, s = x.shape
    dm = W.shape[1]
    xt = x.astype(jnp.int32).T
    nfull_cols = W.shape[0] // LANE * LANE
    wtail = W[nfull_cols:].reshape(-1, LANE)
    w4 = _make_transpose(W.shape[0], dm)(W.T, wtail)
    out = _make_kernel(b, s, dm)(xt, w4)
    return out.transpose(2, 4, 0, 1, 3).reshape(b, s, dm)


# speedup vs baseline: 1.0930x; 1.0515x over previous
import functools

import jax
import jax.numpy as jnp
from jax import lax
from jax.experimental import pallas as pl
from jax.experimental.pallas import tpu as pltpu
from jax.experimental.pallas import tpu_sc as plsc

NJ = 5
LANE = 128


@functools.lru_cache(maxsize=None)
def _make_transpose(n, dm):
    info = plsc.get_sparse_core_info()
    nc, ns = info.num_cores, info.num_subcores
    nw = nc * ns
    nfull = n // LANE
    ntail = n - nfull * LANE
    per_w = nfull // nw
    rem = nfull - per_w * nw
    rpb = LANE * dm // LANE
    mesh = plsc.VectorSubcoreMesh(core_axis_name="c", subcore_axis_name="s")

    @functools.partial(
        pl.kernel,
        mesh=mesh,
        compiler_params=pltpu.CompilerParams(
            use_tc_tiling_on_sc=True, needs_layout_passes=False
        ),
        out_type=jax.ShapeDtypeStruct((n * dm // LANE, LANE), jnp.float32),
        scratch_types=[
            pltpu.VMEM((2, dm, LANE), jnp.float32),
            pltpu.VMEM((2, rpb, LANE), jnp.float32),
            pltpu.SemaphoreType.DMA,
            pltpu.SemaphoreType.DMA,
        ],
    )
    def sc_transpose(vt_hbm, wtail_hbm, out_hbm, vb_v, t_v, sem_r, sem_w):
        wid = lax.axis_index("s") * nc + lax.axis_index("c")
        base = wid * per_w + jnp.minimum(wid, rem)
        nblk = per_w + (wid < rem).astype(jnp.int32)
        iota = lax.iota(jnp.int32, 16)

        def fire_read(c, p):
            pltpu.async_copy(
                vt_hbm.at[:, pl.ds(c * LANE, LANE)], vb_v.at[p], sem_r
            )

        zero = jnp.zeros((16,), jnp.int32)

        def transpose_block(p, nrow):
            @plsc.parallel_loop(0, nrow, 1, unroll=2)
            def tr(i2):
                b4 = zero + i2 * 4
                for h in range(8):
                    base_h = iota * LANE + ((h % 2) * 16 * LANE + h // 2)
                    vals = plsc.load_gather(vb_v.at[p], [zero, base_h + b4])
                    t_v[p, i2, pl.ds(h * 16, 16)] = vals

        fire_read(base, 0)

        def body(i, carry):
            p = i % 2

            @pl.when(i + 1 < nblk)
            def _():
                fire_read(base + i + 1, (i + 1) % 2)

            pltpu.make_async_copy(
                vt_hbm.at[:, pl.ds(0, LANE)], vb_v.at[p], sem_r
            ).wait()

            @pl.when(i >= 2)
            def _():
                pltpu.make_async_copy(
                    vt_hbm.at[:, pl.ds(0, LANE)], t_v.at[p], sem_w
                ).wait()

            transpose_block(p, rpb)
            pltpu.async_copy(
                t_v.at[p], out_hbm.at[pl.ds((base + i) * rpb, rpb)], sem_w
            )
            return carry

        lax.fori_loop(0, nblk, body, 0)
        for _ in range(2):
            pltpu.make_async_copy(
                vt_hbm.at[:, pl.ds(0, LANE)], t_v.at[0], sem_w
            ).wait()

        if ntail:

            @pl.when(wid == nw - 1)
            def _():
                trow = ntail * dm // LANE
                pltpu.sync_copy(wtail_hbm, t_v.at[0, pl.ds(0, trow)])
                pltpu.sync_copy(
                    t_v.at[0, pl.ds(0, trow)],
                    out_hbm.at[pl.ds(nfull * rpb, trow)],
                )

    return sc_transpose


@functools.lru_cache(maxsize=None)
def _make_kernel(b, s, dm):
    info = plsc.get_sparse_core_info()
    nc, ns = info.num_cores, info.num_subcores
    nw = nc * ns
    n_blocks = b // LANE
    cb_per_w = n_blocks // nw
    ngc = s // NJ
    ng = cb_per_w * ngc
    gr = dm // 8
    mesh = plsc.VectorSubcoreMesh(core_axis_name="c", subcore_axis_name="s")

    @functools.partial(
        pl.kernel,
        mesh=mesh,
        compiler_params=pltpu.CompilerParams(
            use_tc_tiling_on_sc=False, needs_layout_passes=False
        ),
        out_type=jax.ShapeDtypeStruct((s, gr, n_blocks, 8, LANE), jnp.float32),
        scratch_types=[
            pltpu.VMEM((2, NJ, LANE), jnp.int32),
            pltpu.VMEM((2, NJ, LANE, dm), jnp.float32),
            pltpu.VMEM((2, gr, NJ, 8, LANE), jnp.float32),
            pltpu.SemaphoreType.DMA,
            pltpu.SemaphoreType.DMA,
            pltpu.SemaphoreType.DMA,
        ],
    )
    def sc_gather(
        xt_hbm, w_hbm, out_hbm, idx_v, rows_v, t_v, sem_a, sem_b, sem_w
    ):
        wid = lax.axis_index("s") * nc + lax.axis_index("c")
        c0 = wid * cb_per_w
        iota = lax.iota(jnp.int32, 16)
        sems = (sem_a, sem_b)

        def coords(g):
            cb = g // ngc
            q = g - cb * ngc
            return c0 + cb, q * NJ

        def stage_fire(g, par):
            cg, j0 = coords(g)
            pltpu.sync_copy(
                xt_hbm.at[pl.ds(j0, NJ), pl.ds(cg * LANE, LANE)], idx_v.at[par]
            )
            for u in range(NJ):
                pltpu.async_copy(
                    w_hbm.at[idx_v.at[par, u]], rows_v.at[par, u], sems[par]
                )

        def drain_gather(par):
            for u in range(NJ):
                pltpu.make_async_copy(
                    w_hbm.at[pl.ds(0, LANE)], rows_v.at[par, u], sems[par]
                ).wait()

        def drain_writes():
            for g_ in range(gr):
                pltpu.make_async_copy(
                    out_hbm.at[pl.ds(0, NJ), 0, 0], t_v.at[0, 0], sem_w
                ).wait()

        def process(g, par):
            cg, j0 = coords(g)

            zero = jnp.zeros((16,), jnp.int32)

            @plsc.parallel_loop(0, NJ * 8, 1, unroll=2)
            def tr(k):
                u = k // 8
                m = k - u * 8
                base = (iota + m * 16) * dm
                for gg in range(gr):
                    for ss in range(8):
                        vals = plsc.load_gather(
                            rows_v.at[par, u], [zero, base + (gg * 8 + ss)]
                        )
                        t_v[par, gg, u, ss, pl.ds(m * 16, 16)] = vals

            for gg in range(gr):
                pltpu.async_copy(
                    t_v.at[par, gg], out_hbm.at[pl.ds(j0, NJ), gg, cg], sem_w
                )

        stage_fire(0, 0)

        def body(s_, carry):
            stage_fire(2 * s_ + 1, 1)
            drain_gather(0)

            @pl.when(s_ > 0)
            def _():
                drain_writes()

            process(2 * s_, 0)
            stage_fire(2 * s_ + 2, 0)
            drain_gather(1)

            @pl.when(s_ > 0)
            def _():
                drain_writes()

            process(2 * s_ + 1, 1)
            return carry

        lax.fori_loop(0, ng // 2 - 1, body, 0)
        stage_fire(ng - 1, 1)
        drain_gather(0)
        drain_writes()
        process(ng - 2, 0)
        drain_gather(1)
        drain_writes()
        process(ng - 1, 1)
        drain_writes()
        drain_writes()

    return sc_gather


def kernel(x, W):
    b, s = x.shape
    dm = W.shape[1]
    xt = x.astype(jnp.int32).T
    nfull_cols = W.shape[0] // LANE * LANE
    wtail = W[nfull_cols:].reshape(-1, LANE)
    w4 = _make_transpose(W.shape[0], dm)(W.T, wtail)
    out = _make_kernel(b, s, dm)(xt, w4.reshape(W.shape[0], dm))
    return out.transpose(2, 4, 0, 1, 3).reshape(b, s, dm)
